# baseline (speedup 1.0000x reference)
import functools

import jax
import jax.numpy as jnp
from jax import lax
from jax.experimental import pallas as pl
from jax.experimental.pallas import tpu as pltpu
from jax.experimental.pallas import tpu_sc as plsc

NUM_ROWS = 1000
DIM = 64
BATCH = 4096
SEQ = 200
PAD = DIM + 1

NC = 2
NS = 16
NW = NC * NS
BPW = BATCH // NW
IDX_PER_W = BPW * SEQ
L = 16
NBLK = BPW // L


@functools.partial(
    pl.kernel,
    out_type=jax.ShapeDtypeStruct((SEQ, DIM, BATCH), jnp.float32),
    mesh=plsc.VectorSubcoreMesh(core_axis_name="c", subcore_axis_name="s"),
    scratch_types=[
        pltpu.VMEM((NUM_ROWS * PAD,), jnp.float32),
        pltpu.VMEM((IDX_PER_W,), jnp.int32),
        pltpu.VMEM((DIM, BPW), jnp.float32),
        pltpu.VMEM((DIM, BPW), jnp.float32),
        pltpu.SemaphoreType.DMA,
        pltpu.SemaphoreType.DMA,
    ],
    compiler_params=pltpu.CompilerParams(use_tc_tiling_on_sc=True,
                                         needs_layout_passes=False),
)
def _lookup(x_hbm, table_hbm, out_hbm, table_v, x_v, buf0, buf1, sem0, sem1):
    wid = lax.axis_index("s") * NC + lax.axis_index("c")

    pltpu.sync_copy(table_hbm, table_v)
    pltpu.sync_copy(x_hbm.at[pl.ds(wid * IDX_PER_W, IDX_PER_W)], x_v)

    lane = lax.iota(jnp.int32, L)
    lane_seq = lane * SEQ

    def build_slab(j, buf):
        for ii in range(NBLK):
            lidx = lane_seq + (ii * L * SEQ) + j
            rows = plsc.load_gather(x_v, [lidx])
            rbase = rows * PAD
            for d in range(DIM):
                v = plsc.load_gather(table_v, [rbase + d])
                buf[d, pl.ds(ii * L, L)] = v

    def start_out(j, buf, sem):
        pltpu.make_async_copy(buf, out_hbm.at[j, :, pl.ds(wid * BPW, BPW)],
                              sem).start()

    def wait_out(buf, sem):
        pltpu.make_async_copy(buf, out_hbm.at[0, :, pl.ds(wid * BPW, BPW)],
                              sem).wait()

    @pl.loop(0, SEQ, step=2)
    def _(j):
        @pl.when(j >= 2)
        def _():
            wait_out(buf0, sem0)

        build_slab(j, buf0)
        start_out(j, buf0, sem0)

        @pl.when(j >= 2)
        def _():
            wait_out(buf1, sem1)

        build_slab(j + 1, buf1)
        start_out(j + 1, buf1, sem1)

    wait_out(buf0, sem0)
    wait_out(buf1, sem1)


def kernel(x, table):
    xf = x.reshape(-1)
    tf = jnp.pad(table, ((0, 0), (0, PAD - DIM))).reshape(-1)
    t = _lookup(xf, tf)
    return jnp.transpose(t, (2, 0, 1))
